# Initial kernel scaffold; baseline (speedup 1.0000x reference)
import jax
import jax.numpy as jnp
from jax.experimental import pallas as pl


def kernel(x, edge_index, conv0_Wl, conv0_bl, conv0_Wr, norm0_g, norm0_b, conv1_Wl, conv1_bl, conv1_Wr, norm1_g, norm1_b, conv2_Wl, conv2_bl, conv2_Wr, norm2_g, norm2_b, reg_W1, reg_b1, reg_W2, reg_b2, reg_W3, reg_b3):
    raise NotImplementedError("write your pallas kernel here")


# speedup vs baseline: 4.6049x; 4.6049x over previous
import functools

import jax
import jax.numpy as jnp
from jax import lax
from jax.experimental import pallas as pl
from jax.experimental.pallas import tpu as pltpu
from jax.experimental.pallas import tpu_sc as plsc

N = 10000
E = 320000
D = 128
H = 128

NC = 2
NS = 16
NW = NC * NS
EPW = E // NW
CH = 80
NIT = EPW // CH
NP = 10240
ROWS_PT = NP // NS
RCH = 128
NRC = ROWS_PT // RCH


def _make_sc_segsum(W):
    mesh = plsc.VectorSubcoreMesh(core_axis_name="c", subcore_axis_name="s")

    @functools.partial(
        pl.kernel,
        mesh=mesh,
        out_type=jax.ShapeDtypeStruct((NC, NP, W), jnp.float32),
        compiler_params=pltpu.CompilerParams(use_tc_tiling_on_sc=False),
        scratch_types=[
            pltpu.VMEM((CH,), jnp.int32),
            pltpu.VMEM((CH,), jnp.int32),
            pltpu.VMEM((CH, W), jnp.float32),
            pltpu.VMEM((RCH, W), jnp.float32),
            pltpu.VMEM_SHARED((NP, W), jnp.float32),
            pltpu.SemaphoreType.DMA,
        ],
    )
    def sc_fn(table, src, dst, out, sidx, didx, rows, zbuf, acc, sem):
        c = lax.axis_index("c")
        s = lax.axis_index("s")
        wid = c * NS + s
        rbase = s * ROWS_PT

        def zrow(i, carry):
            def zcol(j, carry2):
                zbuf[i, pl.ds(j * 16, 16)] = jnp.zeros((16,), jnp.float32)
                return carry2
            return lax.fori_loop(0, W // 16, zcol, carry)
        lax.fori_loop(0, RCH, zrow, 0)
        for k in range(NRC):
            pltpu.sync_copy(zbuf, acc.at[pl.ds(rbase + k * RCH, RCH)])
        plsc.subcore_barrier()

        ebase = wid * EPW

        def body(i, carry):
            off = ebase + i * CH
            pltpu.sync_copy(src.at[pl.ds(off, CH)], sidx)
            pltpu.sync_copy(dst.at[pl.ds(off, CH)], didx)
            pltpu.async_copy(table.at[sidx], rows, sem).wait()
            pltpu.sync_copy(rows, acc.at[didx], add=True)
            return carry
        lax.fori_loop(0, NIT, body, 0)
        plsc.subcore_barrier()

        for k in range(NRC):
            pltpu.sync_copy(acc.at[pl.ds(rbase + k * RCH, RCH)], zbuf)
            pltpu.sync_copy(zbuf, out.at[c].at[pl.ds(rbase + k * RCH, RCH)])

    return sc_fn


_sc_segsum_144 = _make_sc_segsum(H + 16)
_sc_segsum_128 = _make_sc_segsum(H)


def _mm_t(a, w):
    return lax.dot_general(a, w, (((1,), (1,)), ((), ())),
                           preferred_element_type=jnp.float32)


def _layer_norm(z, g, b):
    mu = jnp.mean(z, axis=-1, keepdims=True)
    d = z - mu
    var = jnp.mean(d * d, axis=-1, keepdims=True)
    return g * d * lax.rsqrt(var + 1e-5) + b


BLK = 1000
GRID = N // BLK


def _full(shape):
    nd = len(shape)
    return pl.BlockSpec(shape, lambda i: (0,) * nd)


def _tc_a_body(x_ref, wl_ref, wr_ref, bl_ref, tab_ref, skip_ref):
    h = x_ref[...]
    p = _mm_t(h, wl_ref[...])
    tail = jnp.where(lax.broadcasted_iota(jnp.int32, (BLK, 16), 1) == 0,
                     1.0, 0.0).astype(jnp.float32)
    tab_ref[...] = jnp.concatenate([p, tail], axis=1)
    skip_ref[...] = _mm_t(h, wr_ref[...]) + bl_ref[...]


def _tc_a(x, wl, wr, bl):
    return pl.pallas_call(
        _tc_a_body,
        grid=(GRID,),
        in_specs=[
            pl.BlockSpec((BLK, D), lambda i: (i, 0)),
            _full((H, D)), _full((H, D)), _full((1, H)),
        ],
        out_specs=[
            pl.BlockSpec((BLK, H + 16), lambda i: (i, 0)),
            pl.BlockSpec((BLK, H), lambda i: (i, 0)),
        ],
        out_shape=[
            jax.ShapeDtypeStruct((N, H + 16), jnp.float32),
            jax.ShapeDtypeStruct((N, H), jnp.float32),
        ],
    )(x, wl, wr, bl)


def _tc_b0_body(part_ref, skip_ref, g_ref, b_ref, wl_ref, wr_ref, bl_ref,
                tab_ref, skip2_ref, inv_ref):
    s = part_ref[0] + part_ref[1]
    deg = s[:, H:H + 1]
    inv = 1.0 / jnp.maximum(deg, 1.0)
    z = s[:, :H] * inv + skip_ref[...]
    h = jnp.maximum(_layer_norm(z, g_ref[...], b_ref[...]), 0.0)
    tab_ref[...] = _mm_t(h, wl_ref[...])
    skip2_ref[...] = _mm_t(h, wr_ref[...]) + bl_ref[...]
    inv_ref[...] = jnp.broadcast_to(inv, (BLK, 8))


def _tc_b0(part, skip, g, b, wl, wr, bl):
    return pl.pallas_call(
        _tc_b0_body,
        grid=(GRID,),
        in_specs=[
            pl.BlockSpec((NC, BLK, H + 16), lambda i: (0, i, 0)),
            pl.BlockSpec((BLK, H), lambda i: (i, 0)),
            _full((1, H)), _full((1, H)),
            _full((H, H)), _full((H, H)), _full((1, H)),
        ],
        out_specs=[
            pl.BlockSpec((BLK, H), lambda i: (i, 0)),
            pl.BlockSpec((BLK, H), lambda i: (i, 0)),
            pl.BlockSpec((BLK, 8), lambda i: (i, 0)),
        ],
        out_shape=[
            jax.ShapeDtypeStruct((N, H), jnp.float32),
            jax.ShapeDtypeStruct((N, H), jnp.float32),
            jax.ShapeDtypeStruct((N, 8), jnp.float32),
        ],
    )(part, skip, g, b, wl, wr, bl)


def _tc_b1_body(part_ref, skip_ref, inv_ref, g_ref, b_ref, wl_ref, wr_ref,
                bl_ref, tab_ref, skip2_ref):
    s = part_ref[0] + part_ref[1]
    inv = inv_ref[:, 0:1]
    z = s * inv + skip_ref[...]
    h = jnp.maximum(_layer_norm(z, g_ref[...], b_ref[...]), 0.0)
    tab_ref[...] = _mm_t(h, wl_ref[...])
    skip2_ref[...] = _mm_t(h, wr_ref[...]) + bl_ref[...]


def _tc_b1(part, skip, inv, g, b, wl, wr, bl):
    return pl.pallas_call(
        _tc_b1_body,
        grid=(GRID,),
        in_specs=[
            pl.BlockSpec((NC, BLK, H), lambda i: (0, i, 0)),
            pl.BlockSpec((BLK, H), lambda i: (i, 0)),
            pl.BlockSpec((BLK, 8), lambda i: (i, 0)),
            _full((1, H)), _full((1, H)),
            _full((H, H)), _full((H, H)), _full((1, H)),
        ],
        out_specs=[
            pl.BlockSpec((BLK, H), lambda i: (i, 0)),
            pl.BlockSpec((BLK, H), lambda i: (i, 0)),
        ],
        out_shape=[
            jax.ShapeDtypeStruct((N, H), jnp.float32),
            jax.ShapeDtypeStruct((N, H), jnp.float32),
        ],
    )(part, skip, inv, g, b, wl, wr, bl)


def _tc_c_body(part_ref, skip_ref, inv_ref, g_ref, b_ref, w1_ref, b1_ref,
               w2_ref, b2_ref, w3_ref, b3_ref, out_ref):
    s = part_ref[0] + part_ref[1]
    inv = inv_ref[:, 0:1]
    z = s * inv + skip_ref[...]
    h = jnp.maximum(_layer_norm(z, g_ref[...], b_ref[...]), 0.0)
    a1 = jnp.maximum(_mm_t(h, w1_ref[...]) + b1_ref[...], 0.0)
    a2 = jnp.maximum(_mm_t(a1, w2_ref[...]) + b2_ref[...], 0.0)
    out_ref[...] = _mm_t(a2, w3_ref[...]) + b3_ref[...]


def _tc_c(part, skip, inv, g, b, w1, b1, w2, b2, w3, b3):
    return pl.pallas_call(
        _tc_c_body,
        grid=(GRID,),
        in_specs=[
            pl.BlockSpec((NC, BLK, H), lambda i: (0, i, 0)),
            pl.BlockSpec((BLK, H), lambda i: (i, 0)),
            pl.BlockSpec((BLK, 8), lambda i: (i, 0)),
            _full((1, H)), _full((1, H)),
            _full((H // 2, H)), _full((1, H // 2)),
            _full((H // 4, H // 2)), _full((1, H // 4)),
            _full((8, H // 4)), _full((1, 8)),
        ],
        out_specs=[pl.BlockSpec((BLK, 8), lambda i: (i, 0))],
        out_shape=[jax.ShapeDtypeStruct((N, 8), jnp.float32)],
    )(part, skip, inv, g, b, w1, b1, w2, b2, w3, b3)[0]


def kernel(x, edge_index, conv0_Wl, conv0_bl, conv0_Wr, norm0_g, norm0_b,
           conv1_Wl, conv1_bl, conv1_Wr, norm1_g, norm1_b,
           conv2_Wl, conv2_bl, conv2_Wr, norm2_g, norm2_b,
           reg_W1, reg_b1, reg_W2, reg_b2, reg_W3, reg_b3):
    src = edge_index[0]
    dst = edge_index[1]
    r = lambda v: v.reshape(1, -1)

    tab0, skip0 = _tc_a(x, conv0_Wl, conv0_Wr, r(conv0_bl))
    part0 = _sc_segsum_144(tab0, src, dst)
    tab1, skip1, inv = _tc_b0(part0, skip0, r(norm0_g), r(norm0_b),
                              conv1_Wl, conv1_Wr, r(conv1_bl))
    part1 = _sc_segsum_128(tab1, src, dst)
    tab2, skip2 = _tc_b1(part1, skip1, inv, r(norm1_g), r(norm1_b),
                         conv2_Wl, conv2_Wr, r(conv2_bl))
    part2 = _sc_segsum_128(tab2, src, dst)
    return _tc_c(part2, skip2, inv, r(norm2_g), r(norm2_b),
                 reg_W1, r(reg_b1), reg_W2, r(reg_b2), reg_W3, r(reg_b3))


# speedup vs baseline: 9.4552x; 2.0533x over previous
import functools

import jax
import jax.numpy as jnp
from jax import lax
from jax.experimental import pallas as pl
from jax.experimental.pallas import tpu as pltpu
from jax.experimental.pallas import tpu_sc as plsc

N = 10000
E = 320000
D = 128
H = 128

NC = 2
NS = 16
NW = NC * NS
EPW = E // NW
CH = 25
NIT = EPW // CH
R = 5
G = NIT // R
NP = 10240
ROWS_PT = NP // NS
RCH = 16
NRC = ROWS_PT // RCH
DW = 16


_MESH = plsc.VectorSubcoreMesh(core_axis_name="c", subcore_axis_name="s")


def _zero_fill(buf, nrows, w):
    def zrow(i, carry):
        def zcol(j, carry2):
            buf[i, pl.ds(j * 16, 16)] = jnp.zeros((16,), jnp.float32)
            return carry2
        return lax.fori_loop(0, w // 16, zcol, carry)
    lax.fori_loop(0, nrows, zrow, 0)


def _make_sc_segsum(W):
    mesh = _MESH

    @functools.partial(
        pl.kernel,
        mesh=mesh,
        out_type=jax.ShapeDtypeStruct((NC, NP, W), jnp.float32),
        compiler_params=pltpu.CompilerParams(use_tc_tiling_on_sc=False),
        scratch_types=[
            pltpu.VMEM((NIT, CH), jnp.int32),
            pltpu.VMEM((NIT, CH), jnp.int32),
            [pltpu.VMEM((CH, W), jnp.float32)] * R,
            pltpu.VMEM((RCH, W), jnp.float32),
            pltpu.VMEM_SHARED((NP, W), jnp.float32),
            [pltpu.SemaphoreType.DMA] * R,
            [pltpu.SemaphoreType.DMA] * R,
        ],
    )
    def sc_fn(table, srcr, dstr, out, sidx, didx, rows, zbuf, acc, gsem, ssem):
        c = lax.axis_index("c")
        s = lax.axis_index("s")
        wid = c * NS + s
        rbase = s * ROWS_PT

        pltpu.sync_copy(srcr.at[wid], sidx)
        pltpu.sync_copy(dstr.at[wid], didx)

        _zero_fill(zbuf, RCH, W)
        for k in range(NRC):
            pltpu.sync_copy(zbuf, acc.at[pl.ds(rbase + k * RCH, RCH)])
        plsc.subcore_barrier()

        def gat(b, i):
            return pltpu.make_async_copy(table.at[sidx.at[i]], rows[b], gsem[b])

        def sca(b, i):
            return pltpu.make_async_copy(rows[b], acc.at[didx.at[i]], ssem[b])

        for b in range(R):
            gat(b, b).start()

        def body(g, carry):
            for b in range(R):
                i = g * R + b
                gat(b, i).wait()
                sca(b, i).start(add=True)

            @pl.when(g < G - 1)
            def _():
                for b in range(R):
                    sca(b, g * R + b).wait()
                    gat(b, (g + 1) * R + b).start()
            return carry
        lax.fori_loop(0, G, body, 0)
        for b in range(R):
            sca(b, (G - 1) * R + b).wait()
        plsc.subcore_barrier()

        pltpu.sync_copy(acc.at[pl.ds(rbase, ROWS_PT)],
                        out.at[c].at[pl.ds(rbase, ROWS_PT)])

    return sc_fn


_sc_segsum_128 = _make_sc_segsum(H)


@functools.partial(
    pl.kernel,
    mesh=_MESH,
    out_type=jax.ShapeDtypeStruct((NC, NP, DW), jnp.float32),
    compiler_params=pltpu.CompilerParams(use_tc_tiling_on_sc=False),
    scratch_types=[
        pltpu.VMEM((NIT, CH), jnp.int32),
        pltpu.VMEM((CH, DW), jnp.float32),
        pltpu.VMEM((RCH, DW), jnp.float32),
        pltpu.VMEM_SHARED((NP, DW), jnp.float32),
        pltpu.SemaphoreType.DMA,
    ],
)
def _sc_deg(dstr, out, didx, ones, zbuf, acc, sem):
    c = lax.axis_index("c")
    s = lax.axis_index("s")
    wid = c * NS + s
    rbase = s * ROWS_PT

    pltpu.sync_copy(dstr.at[wid], didx)

    _zero_fill(zbuf, RCH, DW)
    for k in range(NRC):
        pltpu.sync_copy(zbuf, acc.at[pl.ds(rbase + k * RCH, RCH)])

    def orow(i, carry):
        ones[i, pl.ds(0, 16)] = jnp.full((16,), 1.0, jnp.float32)
        return carry
    lax.fori_loop(0, CH, orow, 0)
    plsc.subcore_barrier()

    def sca(i):
        return pltpu.make_async_copy(ones, acc.at[didx.at[i]], sem)

    FB = 8

    def body(g, carry):
        for j in range(FB):
            sca(g * FB + j).start(add=True)
        for j in range(FB):
            sca(g * FB + j).wait()
        return carry
    lax.fori_loop(0, NIT // FB, body, 0)
    plsc.subcore_barrier()

    pltpu.sync_copy(acc.at[pl.ds(rbase, ROWS_PT)],
                    out.at[c].at[pl.ds(rbase, ROWS_PT)])


def _mm_t(a, w):
    return lax.dot_general(a, w, (((1,), (1,)), ((), ())),
                           preferred_element_type=jnp.float32)


def _layer_norm(z, g, b):
    mu = jnp.mean(z, axis=-1, keepdims=True)
    d = z - mu
    var = jnp.mean(d * d, axis=-1, keepdims=True)
    return g * d * lax.rsqrt(var + 1e-5) + b


BLK = 1000
GRID = N // BLK


def _full(shape):
    nd = len(shape)
    return pl.BlockSpec(shape, lambda i: (0,) * nd)


def _tc_a_body(x_ref, wl_ref, wr_ref, bl_ref, tab_ref, skip_ref):
    h = x_ref[...]
    tab_ref[...] = _mm_t(h, wl_ref[...])
    skip_ref[...] = _mm_t(h, wr_ref[...]) + bl_ref[...]


def _tc_a(x, wl, wr, bl):
    return pl.pallas_call(
        _tc_a_body,
        grid=(GRID,),
        in_specs=[
            pl.BlockSpec((BLK, D), lambda i: (i, 0)),
            _full((H, D)), _full((H, D)), _full((1, H)),
        ],
        out_specs=[
            pl.BlockSpec((BLK, H), lambda i: (i, 0)),
            pl.BlockSpec((BLK, H), lambda i: (i, 0)),
        ],
        out_shape=[
            jax.ShapeDtypeStruct((N, H), jnp.float32),
            jax.ShapeDtypeStruct((N, H), jnp.float32),
        ],
    )(x, wl, wr, bl)


def _tc_b0_body(part_ref, pd_ref, skip_ref, g_ref, b_ref, wl_ref, wr_ref,
                bl_ref, tab_ref, skip2_ref, inv_ref):
    s = part_ref[0] + part_ref[1]
    deg = pd_ref[0, :, 0:1] + pd_ref[1, :, 0:1]
    inv = 1.0 / jnp.maximum(deg, 1.0)
    z = s * inv + skip_ref[...]
    h = jnp.maximum(_layer_norm(z, g_ref[...], b_ref[...]), 0.0)
    tab_ref[...] = _mm_t(h, wl_ref[...])
    skip2_ref[...] = _mm_t(h, wr_ref[...]) + bl_ref[...]
    inv_ref[...] = jnp.broadcast_to(inv, (BLK, 8))


def _tc_b0(part, pd, skip, g, b, wl, wr, bl):
    return pl.pallas_call(
        _tc_b0_body,
        grid=(GRID,),
        in_specs=[
            pl.BlockSpec((NC, BLK, H), lambda i: (0, i, 0)),
            pl.BlockSpec((NC, BLK, DW), lambda i: (0, i, 0)),
            pl.BlockSpec((BLK, H), lambda i: (i, 0)),
            _full((1, H)), _full((1, H)),
            _full((H, H)), _full((H, H)), _full((1, H)),
        ],
        out_specs=[
            pl.BlockSpec((BLK, H), lambda i: (i, 0)),
            pl.BlockSpec((BLK, H), lambda i: (i, 0)),
            pl.BlockSpec((BLK, 8), lambda i: (i, 0)),
        ],
        out_shape=[
            jax.ShapeDtypeStruct((N, H), jnp.float32),
            jax.ShapeDtypeStruct((N, H), jnp.float32),
            jax.ShapeDtypeStruct((N, 8), jnp.float32),
        ],
    )(part, pd, skip, g, b, wl, wr, bl)


def _tc_b1_body(part_ref, skip_ref, inv_ref, g_ref, b_ref, wl_ref, wr_ref,
                bl_ref, tab_ref, skip2_ref):
    s = part_ref[0] + part_ref[1]
    inv = inv_ref[:, 0:1]
    z = s * inv + skip_ref[...]
    h = jnp.maximum(_layer_norm(z, g_ref[...], b_ref[...]), 0.0)
    tab_ref[...] = _mm_t(h, wl_ref[...])
    skip2_ref[...] = _mm_t(h, wr_ref[...]) + bl_ref[...]


def _tc_b1(part, skip, inv, g, b, wl, wr, bl):
    return pl.pallas_call(
        _tc_b1_body,
        grid=(GRID,),
        in_specs=[
            pl.BlockSpec((NC, BLK, H), lambda i: (0, i, 0)),
            pl.BlockSpec((BLK, H), lambda i: (i, 0)),
            pl.BlockSpec((BLK, 8), lambda i: (i, 0)),
            _full((1, H)), _full((1, H)),
            _full((H, H)), _full((H, H)), _full((1, H)),
        ],
        out_specs=[
            pl.BlockSpec((BLK, H), lambda i: (i, 0)),
            pl.BlockSpec((BLK, H), lambda i: (i, 0)),
        ],
        out_shape=[
            jax.ShapeDtypeStruct((N, H), jnp.float32),
            jax.ShapeDtypeStruct((N, H), jnp.float32),
        ],
    )(part, skip, inv, g, b, wl, wr, bl)


def _tc_c_body(part_ref, skip_ref, inv_ref, g_ref, b_ref, w1_ref, b1_ref,
               w2_ref, b2_ref, w3_ref, b3_ref, out_ref):
    s = part_ref[0] + part_ref[1]
    inv = inv_ref[:, 0:1]
    z = s * inv + skip_ref[...]
    h = jnp.maximum(_layer_norm(z, g_ref[...], b_ref[...]), 0.0)
    a1 = jnp.maximum(_mm_t(h, w1_ref[...]) + b1_ref[...], 0.0)
    a2 = jnp.maximum(_mm_t(a1, w2_ref[...]) + b2_ref[...], 0.0)
    out_ref[...] = _mm_t(a2, w3_ref[...]) + b3_ref[...]


def _tc_c(part, skip, inv, g, b, w1, b1, w2, b2, w3, b3):
    return pl.pallas_call(
        _tc_c_body,
        grid=(GRID,),
        in_specs=[
            pl.BlockSpec((NC, BLK, H), lambda i: (0, i, 0)),
            pl.BlockSpec((BLK, H), lambda i: (i, 0)),
            pl.BlockSpec((BLK, 8), lambda i: (i, 0)),
            _full((1, H)), _full((1, H)),
            _full((H // 2, H)), _full((1, H // 2)),
            _full((H // 4, H // 2)), _full((1, H // 4)),
            _full((8, H // 4)), _full((1, 8)),
        ],
        out_specs=[pl.BlockSpec((BLK, 8), lambda i: (i, 0))],
        out_shape=[jax.ShapeDtypeStruct((N, 8), jnp.float32)],
    )(part, skip, inv, g, b, w1, b1, w2, b2, w3, b3)[0]


def kernel(x, edge_index, conv0_Wl, conv0_bl, conv0_Wr, norm0_g, norm0_b,
           conv1_Wl, conv1_bl, conv1_Wr, norm1_g, norm1_b,
           conv2_Wl, conv2_bl, conv2_Wr, norm2_g, norm2_b,
           reg_W1, reg_b1, reg_W2, reg_b2, reg_W3, reg_b3):
    src = edge_index[0].reshape(NW, NIT, CH)
    dst = edge_index[1].reshape(NW, NIT, CH)
    r = lambda v: v.reshape(1, -1)

    tab0, skip0 = _tc_a(x, conv0_Wl, conv0_Wr, r(conv0_bl))
    pdeg = _sc_deg(dst)
    part0 = _sc_segsum_128(tab0, src, dst)
    tab1, skip1, inv = _tc_b0(part0, pdeg, skip0, r(norm0_g), r(norm0_b),
                              conv1_Wl, conv1_Wr, r(conv1_bl))
    part1 = _sc_segsum_128(tab1, src, dst)
    tab2, skip2 = _tc_b1(part1, skip1, inv, r(norm1_g), r(norm1_b),
                         conv2_Wl, conv2_Wr, r(conv2_bl))
    part2 = _sc_segsum_128(tab2, src, dst)
    return _tc_c(part2, skip2, inv, r(norm2_g), r(norm2_b),
                 reg_W1, r(reg_b1), reg_W2, r(reg_b2), reg_W3, r(reg_b3))


# speedup vs baseline: 10.5514x; 1.1159x over previous
import functools

import jax
import jax.numpy as jnp
from jax import lax
from jax.experimental import pallas as pl
from jax.experimental.pallas import tpu as pltpu
from jax.experimental.pallas import tpu_sc as plsc

N = 10000
E = 320000
D = 128
H = 128

NC = 2
NS = 16
NW = NC * NS
EPW = E // NW
CH = 50
NIT = EPW // CH
HALF = NIT // 2
R = 5
G2 = HALF // R
NP = 10240
ROWS_PT = NP // NS
RCH = 16
NRC = ROWS_PT // RCH
DW = 16


_MESH = plsc.VectorSubcoreMesh(core_axis_name="c", subcore_axis_name="s")


def _zero_fill(buf, nrows, w):
    def zrow(i, carry):
        def zcol(j, carry2):
            buf[i, pl.ds(j * 16, 16)] = jnp.zeros((16,), jnp.float32)
            return carry2
        return lax.fori_loop(0, w // 16, zcol, carry)
    lax.fori_loop(0, nrows, zrow, 0)


def _make_sc_segsum(W):
    mesh = _MESH

    @functools.partial(
        pl.kernel,
        mesh=mesh,
        out_type=jax.ShapeDtypeStruct((NC, NP, W), jnp.float32),
        compiler_params=pltpu.CompilerParams(use_tc_tiling_on_sc=False),
        scratch_types=[
            pltpu.VMEM((HALF, CH), jnp.int32),
            pltpu.VMEM((HALF, CH), jnp.int32),
            [pltpu.VMEM((CH, W), jnp.float32)] * R,
            pltpu.VMEM((RCH, W), jnp.float32),
            pltpu.VMEM_SHARED((NP, W), jnp.float32),
            [pltpu.SemaphoreType.DMA] * R,
            [pltpu.SemaphoreType.DMA] * R,
        ],
    )
    def sc_fn(table, srcr, dstr, out, sidx, didx, rows, zbuf, acc, gsem, ssem):
        c = lax.axis_index("c")
        s = lax.axis_index("s")
        wid = c * NS + s
        rbase = s * ROWS_PT

        _zero_fill(zbuf, RCH, W)
        for k in range(NRC):
            pltpu.sync_copy(zbuf, acc.at[pl.ds(rbase + k * RCH, RCH)])
        plsc.subcore_barrier()

        def gat(b, i):
            return pltpu.make_async_copy(table.at[sidx.at[i]], rows[b], gsem[b])

        def sca(b, i):
            return pltpu.make_async_copy(rows[b], acc.at[didx.at[i]], ssem[b])

        for h in range(2):
            pltpu.sync_copy(srcr.at[wid].at[pl.ds(h * HALF, HALF)], sidx)
            pltpu.sync_copy(dstr.at[wid].at[pl.ds(h * HALF, HALF)], didx)
            for b in range(R):
                gat(b, b).start()

            def body(g, carry):
                for b in range(R):
                    i = g * R + b
                    gat(b, i).wait()
                    sca(b, i).start(add=True)

                @pl.when(g < G2 - 1)
                def _():
                    for b in range(R):
                        sca(b, g * R + b).wait()
                        gat(b, (g + 1) * R + b).start()
                return carry
            lax.fori_loop(0, G2, body, 0)
            for b in range(R):
                sca(b, (G2 - 1) * R + b).wait()
        plsc.subcore_barrier()

        pltpu.sync_copy(acc.at[pl.ds(rbase, ROWS_PT)],
                        out.at[c].at[pl.ds(rbase, ROWS_PT)])

    return sc_fn


_sc_segsum_128 = _make_sc_segsum(H)


@functools.partial(
    pl.kernel,
    mesh=_MESH,
    out_type=jax.ShapeDtypeStruct((NC, NP, DW), jnp.float32),
    compiler_params=pltpu.CompilerParams(use_tc_tiling_on_sc=False),
    scratch_types=[
        pltpu.VMEM((NIT, CH), jnp.int32),
        pltpu.VMEM((CH, DW), jnp.float32),
        pltpu.VMEM((RCH, DW), jnp.float32),
        pltpu.VMEM_SHARED((NP, DW), jnp.float32),
        pltpu.SemaphoreType.DMA,
    ],
)
def _sc_deg(dstr, out, didx, ones, zbuf, acc, sem):
    c = lax.axis_index("c")
    s = lax.axis_index("s")
    wid = c * NS + s
    rbase = s * ROWS_PT

    pltpu.sync_copy(dstr.at[wid], didx)

    _zero_fill(zbuf, RCH, DW)
    for k in range(NRC):
        pltpu.sync_copy(zbuf, acc.at[pl.ds(rbase + k * RCH, RCH)])

    def orow(i, carry):
        ones[i, pl.ds(0, 16)] = jnp.full((16,), 1.0, jnp.float32)
        return carry
    lax.fori_loop(0, CH, orow, 0)
    plsc.subcore_barrier()

    def sca(i):
        return pltpu.make_async_copy(ones, acc.at[didx.at[i]], sem)

    FB = 8

    def body(g, carry):
        for j in range(FB):
            sca(g * FB + j).start(add=True)
        for j in range(FB):
            sca(g * FB + j).wait()
        return carry
    lax.fori_loop(0, NIT // FB, body, 0)
    plsc.subcore_barrier()

    pltpu.sync_copy(acc.at[pl.ds(rbase, ROWS_PT)],
                    out.at[c].at[pl.ds(rbase, ROWS_PT)])


def _mm_t(a, w):
    return lax.dot_general(a, w, (((1,), (1,)), ((), ())),
                           preferred_element_type=jnp.float32)


def _layer_norm(z, g, b):
    mu = jnp.mean(z, axis=-1, keepdims=True)
    d = z - mu
    var = jnp.mean(d * d, axis=-1, keepdims=True)
    return g * d * lax.rsqrt(var + 1e-5) + b


BLK = 1000
GRID = N // BLK


def _full(shape):
    nd = len(shape)
    return pl.BlockSpec(shape, lambda i: (0,) * nd)


def _tc_a_body(x_ref, wl_ref, wr_ref, bl_ref, tab_ref, skip_ref):
    h = x_ref[...]
    tab_ref[...] = _mm_t(h, wl_ref[...])
    skip_ref[...] = _mm_t(h, wr_ref[...]) + bl_ref[...]


def _tc_a(x, wl, wr, bl):
    return pl.pallas_call(
        _tc_a_body,
        grid=(GRID,),
        in_specs=[
            pl.BlockSpec((BLK, D), lambda i: (i, 0)),
            _full((H, D)), _full((H, D)), _full((1, H)),
        ],
        out_specs=[
            pl.BlockSpec((BLK, H), lambda i: (i, 0)),
            pl.BlockSpec((BLK, H), lambda i: (i, 0)),
        ],
        out_shape=[
            jax.ShapeDtypeStruct((N, H), jnp.float32),
            jax.ShapeDtypeStruct((N, H), jnp.float32),
        ],
    )(x, wl, wr, bl)


def _tc_b0_body(part_ref, pd_ref, skip_ref, g_ref, b_ref, wl_ref, wr_ref,
                bl_ref, tab_ref, skip2_ref, inv_ref):
    s = part_ref[0] + part_ref[1]
    deg = pd_ref[0, :, 0:1] + pd_ref[1, :, 0:1]
    inv = 1.0 / jnp.maximum(deg, 1.0)
    z = s * inv + skip_ref[...]
    h = jnp.maximum(_layer_norm(z, g_ref[...], b_ref[...]), 0.0)
    tab_ref[...] = _mm_t(h, wl_ref[...])
    skip2_ref[...] = _mm_t(h, wr_ref[...]) + bl_ref[...]
    inv_ref[...] = jnp.broadcast_to(inv, (BLK, 8))


def _tc_b0(part, pd, skip, g, b, wl, wr, bl):
    return pl.pallas_call(
        _tc_b0_body,
        grid=(GRID,),
        in_specs=[
            pl.BlockSpec((NC, BLK, H), lambda i: (0, i, 0)),
            pl.BlockSpec((NC, BLK, DW), lambda i: (0, i, 0)),
            pl.BlockSpec((BLK, H), lambda i: (i, 0)),
            _full((1, H)), _full((1, H)),
            _full((H, H)), _full((H, H)), _full((1, H)),
        ],
        out_specs=[
            pl.BlockSpec((BLK, H), lambda i: (i, 0)),
            pl.BlockSpec((BLK, H), lambda i: (i, 0)),
            pl.BlockSpec((BLK, 8), lambda i: (i, 0)),
        ],
        out_shape=[
            jax.ShapeDtypeStruct((N, H), jnp.float32),
            jax.ShapeDtypeStruct((N, H), jnp.float32),
            jax.ShapeDtypeStruct((N, 8), jnp.float32),
        ],
    )(part, pd, skip, g, b, wl, wr, bl)


def _tc_b1_body(part_ref, skip_ref, inv_ref, g_ref, b_ref, wl_ref, wr_ref,
                bl_ref, tab_ref, skip2_ref):
    s = part_ref[0] + part_ref[1]
    inv = inv_ref[:, 0:1]
    z = s * inv + skip_ref[...]
    h = jnp.maximum(_layer_norm(z, g_ref[...], b_ref[...]), 0.0)
    tab_ref[...] = _mm_t(h, wl_ref[...])
    skip2_ref[...] = _mm_t(h, wr_ref[...]) + bl_ref[...]


def _tc_b1(part, skip, inv, g, b, wl, wr, bl):
    return pl.pallas_call(
        _tc_b1_body,
        grid=(GRID,),
        in_specs=[
            pl.BlockSpec((NC, BLK, H), lambda i: (0, i, 0)),
            pl.BlockSpec((BLK, H), lambda i: (i, 0)),
            pl.BlockSpec((BLK, 8), lambda i: (i, 0)),
            _full((1, H)), _full((1, H)),
            _full((H, H)), _full((H, H)), _full((1, H)),
        ],
        out_specs=[
            pl.BlockSpec((BLK, H), lambda i: (i, 0)),
            pl.BlockSpec((BLK, H), lambda i: (i, 0)),
        ],
        out_shape=[
            jax.ShapeDtypeStruct((N, H), jnp.float32),
            jax.ShapeDtypeStruct((N, H), jnp.float32),
        ],
    )(part, skip, inv, g, b, wl, wr, bl)


def _tc_c_body(part_ref, skip_ref, inv_ref, g_ref, b_ref, w1_ref, b1_ref,
               w2_ref, b2_ref, w3_ref, b3_ref, out_ref):
    s = part_ref[0] + part_ref[1]
    inv = inv_ref[:, 0:1]
    z = s * inv + skip_ref[...]
    h = jnp.maximum(_layer_norm(z, g_ref[...], b_ref[...]), 0.0)
    a1 = jnp.maximum(_mm_t(h, w1_ref[...]) + b1_ref[...], 0.0)
    a2 = jnp.maximum(_mm_t(a1, w2_ref[...]) + b2_ref[...], 0.0)
    out_ref[...] = _mm_t(a2, w3_ref[...]) + b3_ref[...]


def _tc_c(part, skip, inv, g, b, w1, b1, w2, b2, w3, b3):
    return pl.pallas_call(
        _tc_c_body,
        grid=(GRID,),
        in_specs=[
            pl.BlockSpec((NC, BLK, H), lambda i: (0, i, 0)),
            pl.BlockSpec((BLK, H), lambda i: (i, 0)),
            pl.BlockSpec((BLK, 8), lambda i: (i, 0)),
            _full((1, H)), _full((1, H)),
            _full((H // 2, H)), _full((1, H // 2)),
            _full((H // 4, H // 2)), _full((1, H // 4)),
            _full((8, H // 4)), _full((1, 8)),
        ],
        out_specs=[pl.BlockSpec((BLK, 8), lambda i: (i, 0))],
        out_shape=[jax.ShapeDtypeStruct((N, 8), jnp.float32)],
    )(part, skip, inv, g, b, w1, b1, w2, b2, w3, b3)[0]


def kernel(x, edge_index, conv0_Wl, conv0_bl, conv0_Wr, norm0_g, norm0_b,
           conv1_Wl, conv1_bl, conv1_Wr, norm1_g, norm1_b,
           conv2_Wl, conv2_bl, conv2_Wr, norm2_g, norm2_b,
           reg_W1, reg_b1, reg_W2, reg_b2, reg_W3, reg_b3):
    src = edge_index[0].reshape(NW, NIT, CH)
    dst = edge_index[1].reshape(NW, NIT, CH)
    r = lambda v: v.reshape(1, -1)

    tab0, skip0 = _tc_a(x, conv0_Wl, conv0_Wr, r(conv0_bl))
    pdeg = _sc_deg(dst)
    part0 = _sc_segsum_128(tab0, src, dst)
    tab1, skip1, inv = _tc_b0(part0, pdeg, skip0, r(norm0_g), r(norm0_b),
                              conv1_Wl, conv1_Wr, r(conv1_bl))
    part1 = _sc_segsum_128(tab1, src, dst)
    tab2, skip2 = _tc_b1(part1, skip1, inv, r(norm1_g), r(norm1_b),
                         conv2_Wl, conv2_Wr, r(conv2_bl))
    part2 = _sc_segsum_128(tab2, src, dst)
    return _tc_c(part2, skip2, inv, r(norm2_g), r(norm2_b),
                 reg_W1, r(reg_b1), reg_W2, r(reg_b2), reg_W3, r(reg_b3))
